# baseline (device time: 13347 ns/iter reference)
import jax
import jax.numpy as jnp
from jax import lax
from jax.experimental import pallas as pl
from jax.experimental.pallas import tpu as pltpu

N_DEV = 4
BLK = 256


def kernel(x, w_mat):
    k_full, k_shard = x.shape
    n = w_mat.shape[1]

    def body(x_ref, w_ref, out_ref, xs_ref, xg_ref, wv_ref,
             send_sems, recv_sems, w_sems, loc_sem):
        my = lax.axis_index("i")

        barrier_sem = pltpu.get_barrier_semaphore()
        for d in range(1, N_DEV):
            peer = lax.rem(my + d, N_DEV)
            pl.semaphore_signal(
                barrier_sem, inc=1,
                device_id=(peer,), device_id_type=pl.DeviceIdType.MESH,
            )

        w_copies = []
        for k in range(N_DEV):
            src_blk = lax.rem(my + k, N_DEV)
            c = pltpu.make_async_copy(
                w_ref.at[pl.ds(src_blk * BLK, BLK), :],
                wv_ref.at[pl.ds(k * BLK, BLK), :],
                w_sems.at[k],
            )
            c.start()
            w_copies.append(c)

        xs_ref[:, :] = x_ref[:, :].astype(jnp.bfloat16)

        loc = pltpu.make_async_copy(
            xs_ref.at[pl.ds(my * BLK, BLK), :], xg_ref.at[0], loc_sem,
        )
        loc.start()

        pl.semaphore_wait(barrier_sem, N_DEV - 1)

        rdmas = []
        for d in range(1, N_DEV):
            dst = lax.rem(my + d, N_DEV)
            rdma = pltpu.make_async_remote_copy(
                src_ref=xs_ref.at[pl.ds(dst * BLK, BLK), :],
                dst_ref=xg_ref.at[N_DEV - d],
                send_sem=send_sems.at[d - 1],
                recv_sem=recv_sems.at[N_DEV - d],
                device_id=(dst,),
                device_id_type=pl.DeviceIdType.MESH,
            )
            rdma.start()
            rdmas.append(rdma)

        def recv_wait(k):
            pltpu.make_async_remote_copy(
                src_ref=xs_ref.at[pl.ds(0, BLK), :],
                dst_ref=xg_ref.at[k],
                send_sem=send_sems.at[0],
                recv_sem=recv_sems.at[k],
                device_id=(my,),
                device_id_type=pl.DeviceIdType.MESH,
            ).wait_recv()

        loc.wait()
        w_copies[0].wait()
        acc = jnp.dot(
            xg_ref[0],
            wv_ref[0:BLK, :].astype(jnp.bfloat16),
            preferred_element_type=jnp.float32,
        )
        for k in (1, 3, 2):
            recv_wait(k)
            w_copies[k].wait()
            acc += jnp.dot(
                xg_ref[k],
                wv_ref[k * BLK:(k + 1) * BLK, :].astype(jnp.bfloat16),
                preferred_element_type=jnp.float32,
            )

        out_ref[:, :] = acc * jax.nn.sigmoid(acc)

        for rdma in rdmas:
            rdma.wait_send()

    return pl.pallas_call(
        body,
        out_shape=jax.ShapeDtypeStruct((BLK, n), jnp.float32),
        in_specs=[
            pl.BlockSpec(memory_space=pltpu.VMEM),
            pl.BlockSpec(memory_space=pltpu.VMEM),
        ],
        out_specs=pl.BlockSpec(memory_space=pltpu.VMEM),
        scratch_shapes=[
            pltpu.VMEM((k_full, k_shard), jnp.bfloat16),
            pltpu.VMEM((N_DEV, BLK, BLK), jnp.bfloat16),
            pltpu.VMEM((k_full, n), jnp.float32),
            pltpu.SemaphoreType.DMA((N_DEV - 1,)),
            pltpu.SemaphoreType.DMA((N_DEV,)),
            pltpu.SemaphoreType.DMA((N_DEV,)),
            pltpu.SemaphoreType.DMA,
        ],
        compiler_params=pltpu.CompilerParams(collective_id=0),
    )(x, w_mat)


# device time: 12561 ns/iter; 1.0626x vs baseline; 1.0626x over previous
import jax
import jax.numpy as jnp
from jax import lax
from jax.experimental import pallas as pl
from jax.experimental.pallas import tpu as pltpu

N_DEV = 4
BLK = 256


def kernel(x, w_mat):
    w_mat = w_mat.astype(jnp.bfloat16)
    k_full, k_shard = x.shape
    n = w_mat.shape[1]

    def body(x_ref, w_ref, out_ref, xs_ref, xg_ref, wv_ref,
             send_sems, recv_sems, w_sems, loc_sem):
        my = lax.axis_index("i")

        barrier_sem = pltpu.get_barrier_semaphore()
        for d in range(1, N_DEV):
            peer = lax.rem(my + d, N_DEV)
            pl.semaphore_signal(
                barrier_sem, inc=1,
                device_id=(peer,), device_id_type=pl.DeviceIdType.MESH,
            )

        w_copies = []
        for k in range(N_DEV):
            src_blk = lax.rem(my + k, N_DEV)
            c = pltpu.make_async_copy(
                w_ref.at[pl.ds(src_blk * BLK, BLK), :],
                wv_ref.at[pl.ds(k * BLK, BLK), :],
                w_sems.at[k],
            )
            c.start()
            w_copies.append(c)

        xs_ref[:, :] = x_ref[:, :].astype(jnp.bfloat16)

        loc = pltpu.make_async_copy(
            xs_ref.at[pl.ds(my * BLK, BLK), :], xg_ref.at[0], loc_sem,
        )
        loc.start()

        pl.semaphore_wait(barrier_sem, N_DEV - 1)

        rdmas = []
        for d in range(1, N_DEV):
            dst = lax.rem(my + d, N_DEV)
            rdma = pltpu.make_async_remote_copy(
                src_ref=xs_ref.at[pl.ds(dst * BLK, BLK), :],
                dst_ref=xg_ref.at[N_DEV - d],
                send_sem=send_sems.at[d - 1],
                recv_sem=recv_sems.at[N_DEV - d],
                device_id=(dst,),
                device_id_type=pl.DeviceIdType.MESH,
            )
            rdma.start()
            rdmas.append(rdma)

        def recv_wait(k):
            pltpu.make_async_remote_copy(
                src_ref=xs_ref.at[pl.ds(0, BLK), :],
                dst_ref=xg_ref.at[k],
                send_sem=send_sems.at[0],
                recv_sem=recv_sems.at[k],
                device_id=(my,),
                device_id_type=pl.DeviceIdType.MESH,
            ).wait_recv()

        loc.wait()
        w_copies[0].wait()
        acc = jnp.dot(
            xg_ref[0],
            wv_ref[0:BLK, :],
            preferred_element_type=jnp.float32,
        )
        for k in (1, 3, 2):
            recv_wait(k)
            w_copies[k].wait()
            acc += jnp.dot(
                xg_ref[k],
                wv_ref[k * BLK:(k + 1) * BLK, :],
                preferred_element_type=jnp.float32,
            )

        out_ref[:, :] = acc * jax.nn.sigmoid(acc)

        for rdma in rdmas:
            rdma.wait_send()

    return pl.pallas_call(
        body,
        out_shape=jax.ShapeDtypeStruct((BLK, n), jnp.float32),
        in_specs=[
            pl.BlockSpec(memory_space=pltpu.VMEM),
            pl.BlockSpec(memory_space=pltpu.VMEM),
        ],
        out_specs=pl.BlockSpec(memory_space=pltpu.VMEM),
        scratch_shapes=[
            pltpu.VMEM((k_full, k_shard), jnp.bfloat16),
            pltpu.VMEM((N_DEV, BLK, BLK), jnp.bfloat16),
            pltpu.VMEM((k_full, n), jnp.bfloat16),
            pltpu.SemaphoreType.DMA((N_DEV - 1,)),
            pltpu.SemaphoreType.DMA((N_DEV,)),
            pltpu.SemaphoreType.DMA((N_DEV,)),
            pltpu.SemaphoreType.DMA,
        ],
        compiler_params=pltpu.CompilerParams(collective_id=0),
    )(x, w_mat)
